# baseline (speedup 1.0000x reference)
import functools

import numpy as np
import jax
from jax import lax
import jax.numpy as jnp
from jax.experimental import pallas as pl
from jax.experimental.pallas import tpu as pltpu
from jax.experimental.pallas import tpu_sc as plsc

_NUM_CLASSES = 395
_K_REGIONS = 32
_K_SHOTS = 4
_C_FEAT = 96
_B = 128
_HW = 64 * 64
_F = _C_FEAT * _K_REGIONS


def _rand_fn(cs):
    return jax.vmap(
        lambda c: jax.random.normal(
            jax.random.fold_in(jax.random.key(1), c), (_F,), dtype=jnp.float32
        )
        * 0.01
    )(cs)


def _try_eager_rand():
    try:
        return np.asarray(_rand_fn(jnp.arange(_NUM_CLASSES, dtype=jnp.int32)))
    except Exception:
        return None


_RAND = _try_eager_rand()


def _get_rand():
    if _RAND is not None:
        return jnp.asarray(_RAND)
    return _rand_fn(jnp.arange(_NUM_CLASSES, dtype=jnp.int32))


def _topk_body(cam_ref, out_ref):
    val = cam_ref[...]
    col = jax.lax.broadcasted_iota(jnp.int32, (_B, _HW), 1)
    col_k = jax.lax.broadcasted_iota(jnp.int32, (_B, _K_REGIONS), 1)

    def body(j, carry):
        val, acc = carry
        m = jnp.max(val, axis=1, keepdims=True)
        idx = jnp.min(jnp.where(val == m, col, _HW), axis=1, keepdims=True)
        acc = jnp.where(col_k == j, idx, acc)
        val = jnp.where(col == idx, -jnp.inf, val)
        return val, acc

    _, acc = jax.lax.fori_loop(
        0, _K_REGIONS, body, (val, jnp.zeros((_B, _K_REGIONS), jnp.int32))
    )
    out_ref[...] = acc


_NW = 32
_B_PER_W = _B // _NW
_GRAN = 16
_N_DMA = _F // 128
_G16 = _F // 16


def _sc_gather_body(fm_hbm, reg_hbm, out_hbm, reg_v, idx_v, row_v, sem):
    wid = lax.axis_index("s") * 2 + lax.axis_index("c")
    base_b = wid * _B_PER_W
    pltpu.sync_copy(reg_hbm.at[pl.ds(base_b, _B_PER_W)], reg_v)

    for bb in range(_B_PER_W):
        b = base_b + bb

        def idx_body(k, _):
            for g in range(8):
                i = k * 8 + g
                c = i >> 1
                jbase = (i & 1) * 16
                hw = reg_v[bb, pl.ds(jbase, 16)]
                idx_v[k, pl.ds(g * 16, 16)] = (b * _C_FEAT + c) * _HW + hw
            return 0

        lax.fori_loop(0, _N_DMA, idx_body, 0)

        copies = [
            pltpu.async_copy(
                fm_hbm.at[idx_v.at[k]], row_v.at[pl.ds(k * 128, 128)], sem
            )
            for k in range(_N_DMA)
        ]
        for cp in copies:
            cp.wait()
        pltpu.sync_copy(row_v, out_hbm.at[pl.ds(b * _F, _F)])


def _sc_gather(fm3, regions):
    fm_flat = fm3.reshape(_B * _C_FEAT * _HW)
    mesh = plsc.VectorSubcoreMesh(core_axis_name="c", subcore_axis_name="s")
    out_flat = pl.kernel(
        _sc_gather_body,
        out_type=jax.ShapeDtypeStruct((_B * _F,), jnp.float32),
        mesh=mesh,
        scratch_types=[
            pltpu.VMEM((_B_PER_W, _K_REGIONS), jnp.int32),
            pltpu.VMEM((_N_DMA, 128), jnp.int32),
            pltpu.VMEM((_F,), jnp.float32),
            pltpu.SemaphoreType.DMA,
        ],
    )(fm_flat, regions)
    return out_flat.reshape(_B, _F)


def _mean_body(labels_ref, feat_ref, p0_ref, rand_ref, counts0_ref, out_ref):
    labels = labels_ref[...]
    cls = jax.lax.broadcasted_iota(jnp.int32, (_NUM_CLASSES, _B), 0)
    mask = (labels == cls).astype(jnp.float32)
    tri = (
        jax.lax.broadcasted_iota(jnp.int32, (_B, _B), 0)
        <= jax.lax.broadcasted_iota(jnp.int32, (_B, _B), 1)
    ).astype(jnp.float32)
    rank = jnp.dot(mask, tri, preferred_element_type=jnp.float32)
    sel = mask * (rank < _K_SHOTS + 0.5)
    n = jnp.sum(mask, axis=1, keepdims=True)
    msum = jnp.dot(sel, feat_ref[...], preferred_element_type=jnp.float32)
    denom = jnp.maximum(jnp.minimum(n, float(_K_SHOTS)), 1.0)
    mean = msum / denom
    p0 = p0_ref[...]
    fallback = jnp.where(counts0_ref[...] == 0.0, rand_ref[...], p0)
    bp = jnp.where(n > 0.0, mean, fallback)
    new = 0.9 * p0 + 0.1 * bp
    norm = jnp.sqrt(jnp.sum(new * new, axis=1, keepdims=True))
    out_ref[...] = new / (norm + 1e-8)


def kernel(cam, feature_map, labels, prototypes, counts):
    cam2 = cam.reshape(_B, _HW)
    regions = pl.pallas_call(
        _topk_body,
        out_shape=jax.ShapeDtypeStruct((_B, _K_REGIONS), jnp.int32),
    )(cam2)

    features = _sc_gather(feature_map.reshape(_B, _C_FEAT, _HW), regions)

    out = pl.pallas_call(
        _mean_body,
        out_shape=jax.ShapeDtypeStruct((_NUM_CLASSES, _F), jnp.float32),
    )(
        labels.reshape(1, _B),
        features,
        prototypes[:, 0],
        _get_rand(),
        counts[:, 0:1],
    )
    return out


# speedup vs baseline: 1.0064x; 1.0064x over previous
import functools

import numpy as np
import jax
from jax import lax
import jax.numpy as jnp
from jax.experimental import pallas as pl
from jax.experimental.pallas import tpu as pltpu
from jax.experimental.pallas import tpu_sc as plsc

_NUM_CLASSES = 395
_K_REGIONS = 32
_K_SHOTS = 4
_C_FEAT = 96
_B = 128
_HW = 64 * 64
_F = _C_FEAT * _K_REGIONS


def _rand_fn(cs):
    return jax.vmap(
        lambda c: jax.random.normal(
            jax.random.fold_in(jax.random.key(1), c), (_F,), dtype=jnp.float32
        )
        * 0.01
    )(cs)


def _try_eager_rand():
    try:
        return np.asarray(_rand_fn(jnp.arange(_NUM_CLASSES, dtype=jnp.int32)))
    except Exception:
        return None


_RAND = _try_eager_rand()


def _get_rand():
    if _RAND is not None:
        return jnp.asarray(_RAND)
    return _rand_fn(jnp.arange(_NUM_CLASSES, dtype=jnp.int32))


def _topk_body(cam_ref, out_ref):
    val = cam_ref[...]
    col = jax.lax.broadcasted_iota(jnp.int32, (_B, _HW), 1)
    col_k = jax.lax.broadcasted_iota(jnp.int32, (_B, _K_REGIONS), 1)

    def body(j, carry):
        val, acc = carry
        m = jnp.max(val, axis=1, keepdims=True)
        idx = jnp.min(jnp.where(val == m, col, _HW), axis=1, keepdims=True)
        acc = jnp.where(col_k == j, idx, acc)
        val = jnp.where(col == idx, -jnp.inf, val)
        return val, acc

    _, acc = jax.lax.fori_loop(
        0, _K_REGIONS, body, (val, jnp.zeros((_B, _K_REGIONS), jnp.int32))
    )
    out_ref[...] = acc


_NW = 32
_B_PER_W = _B // _NW
_GRAN = 16
_N_DMA = _F // 128
_G16 = _F // 16


def _sc_gather_body(fm_hbm, reg_hbm, out_hbm, reg_v, idx_v, row_v, sem):
    wid = lax.axis_index("s") * 2 + lax.axis_index("c")
    base_b = wid * _B_PER_W
    pltpu.sync_copy(reg_hbm.at[pl.ds(base_b, _B_PER_W)], reg_v)

    def idx_body(k, _):
        bb = k // _N_DMA
        for g in range(8):
            i = (k % _N_DMA) * 8 + g
            c = i >> 1
            jbase = (i & 1) * 16
            hw = reg_v[bb, pl.ds(jbase, 16)]
            idx_v[k, pl.ds(g * 16, 16)] = ((base_b + bb) * _C_FEAT + c) * _HW + hw
        return 0

    lax.fori_loop(0, _B_PER_W * _N_DMA, idx_body, 0)

    copies = [
        pltpu.async_copy(fm_hbm.at[idx_v.at[k]], row_v.at[pl.ds(k * 128, 128)], sem)
        for k in range(_B_PER_W * _N_DMA)
    ]
    for cp in copies:
        cp.wait()
    pltpu.sync_copy(row_v, out_hbm.at[pl.ds(base_b * _F, _B_PER_W * _F)])


def _sc_gather(fm3, regions):
    fm_flat = fm3.reshape(_B * _C_FEAT * _HW)
    mesh = plsc.VectorSubcoreMesh(core_axis_name="c", subcore_axis_name="s")
    out_flat = pl.kernel(
        _sc_gather_body,
        out_type=jax.ShapeDtypeStruct((_B * _F,), jnp.float32),
        mesh=mesh,
        scratch_types=[
            pltpu.VMEM((_B_PER_W, _K_REGIONS), jnp.int32),
            pltpu.VMEM((_B_PER_W * _N_DMA, 128), jnp.int32),
            pltpu.VMEM((_B_PER_W * _F,), jnp.float32),
            pltpu.SemaphoreType.DMA,
        ],
    )(fm_flat, regions)
    return out_flat.reshape(_B, _F)


def _mean_body(labels_ref, feat_ref, p0_ref, rand_ref, counts0_ref, out_ref):
    labels = labels_ref[...]
    cls = jax.lax.broadcasted_iota(jnp.int32, (_NUM_CLASSES, _B), 0)
    mask = (labels == cls).astype(jnp.float32)
    tri = (
        jax.lax.broadcasted_iota(jnp.int32, (_B, _B), 0)
        <= jax.lax.broadcasted_iota(jnp.int32, (_B, _B), 1)
    ).astype(jnp.float32)
    rank = jnp.dot(mask, tri, preferred_element_type=jnp.float32)
    sel = mask * (rank < _K_SHOTS + 0.5)
    n = jnp.sum(mask, axis=1, keepdims=True)
    msum = jnp.dot(sel, feat_ref[...], preferred_element_type=jnp.float32)
    denom = jnp.maximum(jnp.minimum(n, float(_K_SHOTS)), 1.0)
    mean = msum / denom
    p0 = p0_ref[...]
    fallback = jnp.where(counts0_ref[...] == 0.0, rand_ref[...], p0)
    bp = jnp.where(n > 0.0, mean, fallback)
    new = 0.9 * p0 + 0.1 * bp
    norm = jnp.sqrt(jnp.sum(new * new, axis=1, keepdims=True))
    out_ref[...] = new / (norm + 1e-8)


def kernel(cam, feature_map, labels, prototypes, counts):
    cam2 = cam.reshape(_B, _HW)
    regions = pl.pallas_call(
        _topk_body,
        out_shape=jax.ShapeDtypeStruct((_B, _K_REGIONS), jnp.int32),
    )(cam2)

    features = _sc_gather(feature_map.reshape(_B, _C_FEAT, _HW), regions)

    out = pl.pallas_call(
        _mean_body,
        out_shape=jax.ShapeDtypeStruct((_NUM_CLASSES, _F), jnp.float32),
    )(
        labels.reshape(1, _B),
        features,
        prototypes[:, 0],
        _get_rand(),
        counts[:, 0:1],
    )
    return out
